# Initial kernel scaffold; baseline (speedup 1.0000x reference)
import jax
import jax.numpy as jnp
from jax.experimental import pallas as pl


def kernel(nf, ef, edge_index, W_node, W_edge, bias_n, bias_e, W_dense, b_dense):
    raise NotImplementedError("write your pallas kernel here")


# speedup vs baseline: 4.7871x; 4.7871x over previous
import functools
import jax
import jax.numpy as jnp
from jax import lax
from jax.experimental import pallas as pl
from jax.experimental.pallas import tpu as pltpu
from jax.experimental.pallas import tpu_sc as plsc

N = 10000
E = 320000
D = 128

NC = 2
NS = 16
NW = NC * NS

E_PER_TILE = E // NW
SCAT_CH = 200
SCAT_ITERS = E_PER_TILE // SCAT_CH
GATH_CH = 200
GATH_ITERS = E_PER_TILE // GATH_CH
N_PAD = 10240
N_PER_TILE = N_PAD // NS


def _nf2_body(nf_ref, w_ref, b_ref, o_ref):
    x = jnp.dot(nf_ref[...], w_ref[...], preferred_element_type=jnp.float32)
    o_ref[...] = jnp.maximum(x + b_ref[...][None, :], 0.0)


def _nf2(nf, W_node, bias_n):
    blk = 2048
    return pl.pallas_call(
        _nf2_body,
        grid=(N_PAD // blk,),
        in_specs=[
            pl.BlockSpec((blk, D), lambda i: (i, 0)),
            pl.BlockSpec((D, D), lambda i: (0, 0)),
            pl.BlockSpec((D,), lambda i: (0,)),
        ],
        out_specs=pl.BlockSpec((blk, D), lambda i: (i, 0)),
        out_shape=jax.ShapeDtypeStruct((N_PAD, D), jnp.float32),
    )(nf, W_node, bias_n)


def _m_body(ef_ref, we_ref, wd_ref, bv_ref, o_ref):
    wc = jnp.dot(we_ref[...], wd_ref[...][:, :D].T,
                 preferred_element_type=jnp.float32)
    x = jnp.dot(ef_ref[...], wc, preferred_element_type=jnp.float32)
    o_ref[...] = x + bv_ref[...][None, :]


def _m_edges(ef, W_edge, W_dense, bvec):
    blk = 2560
    return pl.pallas_call(
        _m_body,
        grid=(E // blk,),
        in_specs=[
            pl.BlockSpec((blk, D), lambda i: (i, 0)),
            pl.BlockSpec((D, D), lambda i: (0, 0)),
            pl.BlockSpec((D, 2 * D), lambda i: (0, 0)),
            pl.BlockSpec((D,), lambda i: (0,)),
        ],
        out_specs=pl.BlockSpec((blk, D), lambda i: (i, 0)),
        out_shape=jax.ShapeDtypeStruct((E, D), jnp.float32),
    )(ef, W_edge, W_dense, bvec)


def _tables_body(sp_ref, dp_ref, nf2_ref, we_ref, wd_ref, ab_ref, bh_ref):
    i = pl.program_id(0)
    blk = ab_ref.shape[0]
    S = sp_ref[0] + sp_ref[1]
    deg = dp_ref[0, pl.ds(i * blk, blk)] + dp_ref[1, pl.ds(i * blk, blk)]
    nb = jnp.dot(S, we_ref[...], preferred_element_type=jnp.float32)
    nb = nb / jnp.maximum(deg, 1.0)[:, None]
    bh = 0.5 * jnp.dot(nf2_ref[...], wd_ref[...][:, D:].T,
                       preferred_element_type=jnp.float32)
    ab = jnp.dot(nb, wd_ref[...][:, :D].T,
                 preferred_element_type=jnp.float32) + bh
    ab_ref[...] = ab
    bh_ref[...] = bh


def _tables(S_part, deg_part, nf2, W_edge, W_dense):
    blk = 2048
    return pl.pallas_call(
        _tables_body,
        grid=(N_PAD // blk,),
        in_specs=[
            pl.BlockSpec((NC, blk, D), lambda i: (0, i, 0)),
            pl.BlockSpec((NC, N_PAD), lambda i: (0, 0)),
            pl.BlockSpec((blk, D), lambda i: (i, 0)),
            pl.BlockSpec((D, D), lambda i: (0, 0)),
            pl.BlockSpec((D, 2 * D), lambda i: (0, 0)),
        ],
        out_specs=[
            pl.BlockSpec((blk, D), lambda i: (i, 0)),
            pl.BlockSpec((blk, D), lambda i: (i, 0)),
        ],
        out_shape=[
            jax.ShapeDtypeStruct((N_PAD, D), jnp.float32),
            jax.ShapeDtypeStruct((N_PAD, D), jnp.float32),
        ],
    )(S_part, deg_part, nf2, W_edge, W_dense)


def _scatter_tec(ef_hbm, dst_hbm, zS_hbm, zdeg_hbm, S_out, deg_out,
                 S_acc, deg_acc, ef_buf, idx_buf, ones_buf):
    c = lax.axis_index("c")
    s = lax.axis_index("s")

    pltpu.sync_copy(zS_hbm.at[pl.ds(s * N_PER_TILE, N_PER_TILE)],
                    S_acc.at[pl.ds(s * N_PER_TILE, N_PER_TILE)])

    @pl.when(s == 0)
    def _():
        pltpu.sync_copy(zdeg_hbm, deg_acc)

    for j in range(SCAT_CH // 16):
        ones_buf[pl.ds(j * 16, 16)] = jnp.ones((16,), jnp.float32)

    plsc.subcore_barrier()

    base = (c * NS + s) * E_PER_TILE

    def step(i, carry):
        off = base + i * SCAT_CH
        pltpu.sync_copy(dst_hbm.at[pl.ds(off, SCAT_CH)], idx_buf)
        pltpu.sync_copy(ef_hbm.at[pl.ds(off, SCAT_CH)], ef_buf)
        pltpu.sync_copy(ef_buf, S_acc.at[idx_buf], add=True)
        pltpu.sync_copy(ones_buf, deg_acc.at[idx_buf], add=True)
        return carry

    lax.fori_loop(0, SCAT_ITERS, step, 0)

    plsc.subcore_barrier()

    pltpu.sync_copy(S_acc.at[pl.ds(s * N_PER_TILE, N_PER_TILE)],
                    S_out.at[c].at[pl.ds(s * N_PER_TILE, N_PER_TILE)])

    @pl.when(s == 0)
    def _():
        pltpu.sync_copy(deg_acc, deg_out.at[c])


def _scatter(ef, dst, zS, zdeg):
    mesh = plsc.VectorSubcoreMesh(core_axis_name="c", subcore_axis_name="s")
    return pl.kernel(
        _scatter_tec,
        out_type=[
            jax.ShapeDtypeStruct((NC, N_PAD, D), jnp.float32),
            jax.ShapeDtypeStruct((NC, N_PAD), jnp.float32),
        ],
        mesh=mesh,
        scratch_types=[
            pltpu.VMEM_SHARED((N_PAD, D), jnp.float32),
            pltpu.VMEM_SHARED((N_PAD,), jnp.float32),
            pltpu.VMEM((SCAT_CH, D), jnp.float32),
            pltpu.VMEM((SCAT_CH,), jnp.int32),
            pltpu.VMEM((SCAT_CH,), jnp.float32),
        ],
    )(ef, dst, zS, zdeg)


def _final_tec(m_hbm, ab_hbm, bh_hbm, dst_hbm, src_hbm, out_hbm,
               buf_m, buf_a, buf_b, idx_d, idx_s, sem_a, sem_b):
    c = lax.axis_index("c")
    s = lax.axis_index("s")
    base = (c * NS + s) * E_PER_TILE

    def step(i, carry):
        off = base + i * GATH_CH
        pltpu.sync_copy(dst_hbm.at[pl.ds(off, GATH_CH)], idx_d)
        pltpu.sync_copy(src_hbm.at[pl.ds(off, GATH_CH)], idx_s)
        cp_a = pltpu.async_copy(ab_hbm.at[idx_d], buf_a, sem_a)
        cp_b = pltpu.async_copy(bh_hbm.at[idx_s], buf_b, sem_b)
        pltpu.sync_copy(m_hbm.at[pl.ds(off, GATH_CH)], buf_m)
        cp_a.wait()
        cp_b.wait()

        def row(r, rc):
            for j in range(D // 16):
                sl = pl.ds(j * 16, 16)
                x = buf_m[r, sl] + buf_a[r, sl] + buf_b[r, sl]
                buf_m[r, sl] = jnp.maximum(x, 0.0)
            return rc

        lax.fori_loop(0, GATH_CH, row, 0)
        pltpu.sync_copy(buf_m, out_hbm.at[pl.ds(off, GATH_CH)])
        return carry

    lax.fori_loop(0, GATH_ITERS, step, 0)


def _final(M, AB, Bh, dst, src):
    mesh = plsc.VectorSubcoreMesh(core_axis_name="c", subcore_axis_name="s")
    return pl.kernel(
        _final_tec,
        out_type=jax.ShapeDtypeStruct((E, D), jnp.float32),
        mesh=mesh,
        scratch_types=[
            pltpu.VMEM((GATH_CH, D), jnp.float32),
            pltpu.VMEM((GATH_CH, D), jnp.float32),
            pltpu.VMEM((GATH_CH, D), jnp.float32),
            pltpu.VMEM((GATH_CH,), jnp.int32),
            pltpu.VMEM((GATH_CH,), jnp.int32),
            pltpu.SemaphoreType.DMA,
            pltpu.SemaphoreType.DMA,
        ],
    )(M, AB, Bh, dst, src)


@jax.jit
def kernel(nf, ef, edge_index, W_node, W_edge, bias_n, bias_e, W_dense, b_dense):
    src = edge_index[0]
    dst = edge_index[1]
    zS = jnp.zeros((N_PAD, D), jnp.float32)
    zdeg = jnp.zeros((N_PAD,), jnp.float32)
    nf_pad = jnp.concatenate([nf, jnp.zeros((N_PAD - N, D), jnp.float32)], axis=0)

    nf2p = _nf2(nf_pad, W_node, bias_n)
    S_part, deg_part = _scatter(ef, dst, zS, zdeg)
    AB, Bh = _tables(S_part, deg_part, nf2p, W_edge, W_dense)
    M = _m_edges(ef, W_edge, W_dense, b_dense + bias_e)
    ef2 = _final(M, AB, Bh, dst, src)
    return (nf2p[:N], ef2)
